# baseline (speedup 1.0000x reference)
import functools

import jax
import jax.numpy as jnp
from jax import lax
from jax.experimental import pallas as pl
from jax.experimental.pallas import tpu as pltpu
from jax.experimental.pallas import tpu_sc as plsc

B, S, D = 4, 2048, 1024
NW = 32
S_PER_W = S // NW
CH = 8
CHW = CH * D
N_CH = S_PER_W // CH


def _sc_body(x_hbm, pos_hbm, out_hbm, pos_v, buf_v, sem):
    wid = lax.axis_index("s") * 2 + lax.axis_index("c")
    s0 = wid * S_PER_W
    pltpu.sync_copy(pos_hbm.at[pl.ds(s0 * D, S_PER_W * D)], pos_v)

    def add_chunk(k_base):
        def body(i, _):
            off = i * 16
            buf_v[pl.ds(off, 16)] = buf_v[pl.ds(off, 16)] + pos_v[
                pl.ds(k_base + off, 16)
            ]
            return 0

        lax.fori_loop(0, CHW // 16, body, 0)

    for b in range(B):
        for k in range(N_CH):
            src = b * S * D + (s0 + k * CH) * D
            pltpu.sync_copy(x_hbm.at[pl.ds(src, CHW)], buf_v)
            add_chunk(k * CHW)
            pltpu.sync_copy(buf_v, out_hbm.at[pl.ds(src, CHW)])


def kernel(x, pos_weight):
    mesh = plsc.VectorSubcoreMesh(core_axis_name="c", subcore_axis_name="s")
    k = functools.partial(
        pl.kernel,
        mesh=mesh,
        out_type=jax.ShapeDtypeStruct((B * S * D,), jnp.float32),
        scratch_types=[
            pltpu.VMEM((S_PER_W * D,), jnp.float32),
            pltpu.VMEM((CHW,), jnp.float32),
            pltpu.SemaphoreType.DMA,
        ],
    )(_sc_body)
    out = k(x.reshape(-1), pos_weight.reshape(-1))
    return out.reshape(B, S, D)


# speedup vs baseline: 1.1990x; 1.1990x over previous
import functools

import jax
import jax.numpy as jnp
from jax import lax
from jax.experimental import pallas as pl
from jax.experimental.pallas import tpu as pltpu
from jax.experimental.pallas import tpu_sc as plsc

B, S, D = 4, 2048, 1024
NW = 32
S_PER_W = S // NW
CH = 16
CHW = CH * D
N_CH = S_PER_W // CH
T = B * N_CH


def _chunk_off(wid, t):
    b = t // N_CH
    k = t % N_CH
    return b * S * D + (wid * S_PER_W + k * CH) * D


def _sc_body(x_hbm, pos_hbm, out_hbm, pos_v, buf_v, sem_in, sem_out):
    wid = lax.axis_index("s") * 2 + lax.axis_index("c")
    s0 = wid * S_PER_W
    pltpu.sync_copy(pos_hbm.at[pl.ds(s0 * D, S_PER_W * D)], pos_v)

    def add_chunk(buf, k_base):
        def body(i, _):
            base = i * 128
            for u in range(8):
                off = base + u * 16
                buf[pl.ds(off, 16)] = buf[pl.ds(off, 16)] + pos_v[
                    pl.ds(k_base + off, 16)
                ]
            return 0

        lax.fori_loop(0, CHW // 128, body, 0)

    in_h = [None, None]
    out_h = [None, None]
    for t in range(T):
        j = t % 2
        if t >= 2 and out_h[j] is not None:
            out_h[j].wait()
        in_h[j] = pltpu.async_copy(
            x_hbm.at[pl.ds(_chunk_off(wid, t), CHW)], buf_v.at[j], sem_in.at[j]
        )
        if t >= 1:
            jp = (t - 1) % 2
            in_h[jp].wait()
            add_chunk(buf_v.at[jp], ((t - 1) % N_CH) * CHW)
            out_h[jp] = pltpu.async_copy(
                buf_v.at[jp],
                out_hbm.at[pl.ds(_chunk_off(wid, t - 1), CHW)],
                sem_out.at[jp],
            )
    jl = (T - 1) % 2
    in_h[jl].wait()
    add_chunk(buf_v.at[jl], ((T - 1) % N_CH) * CHW)
    pltpu.sync_copy(buf_v.at[jl], out_hbm.at[pl.ds(_chunk_off(wid, T - 1), CHW)])
    jp = (T - 2) % 2
    if out_h[jp] is not None:
        out_h[jp].wait()


def kernel(x, pos_weight):
    mesh = plsc.VectorSubcoreMesh(core_axis_name="c", subcore_axis_name="s")
    k = functools.partial(
        pl.kernel,
        mesh=mesh,
        out_type=jax.ShapeDtypeStruct((B * S * D,), jnp.float32),
        scratch_types=[
            pltpu.VMEM((S_PER_W * D,), jnp.float32),
            pltpu.VMEM((2, CHW), jnp.float32),
            pltpu.SemaphoreType.DMA((2,)),
            pltpu.SemaphoreType.DMA((2,)),
        ],
    )(_sc_body)
    out = k(x.reshape(-1), pos_weight.reshape(-1))
    return out.reshape(B, S, D)


# speedup vs baseline: 1.7866x; 1.4900x over previous
import functools

import jax
import jax.numpy as jnp
from jax import lax
from jax.experimental import pallas as pl
from jax.experimental.pallas import tpu as pltpu
from jax.experimental.pallas import tpu_sc as plsc

B, S, D = 4, 2048, 1024
NW = 32
S_PER_W = S // NW
CH = 16
N_CH = S_PER_W // CH
T = B * N_CH


def _sc_body(x_hbm, pos_hbm, out_hbm, pos_v, buf_v, sem_in, sem_out):
    wid = lax.axis_index("s") * 2 + lax.axis_index("c")
    s0 = wid * S_PER_W
    pltpu.sync_copy(pos_hbm.at[pl.ds(s0, S_PER_W)], pos_v)

    def add_chunk(buf, k):
        def body(r, _):
            pr = k * CH + r

            def col(c, _):
                base = c * 128
                for u in range(8):
                    off = base + u * 16
                    buf[r, pl.ds(off, 16)] = buf[r, pl.ds(off, 16)] + pos_v[
                        pr, pl.ds(off, 16)
                    ]
                return 0

            lax.fori_loop(0, D // 128, col, 0)
            return 0

        lax.fori_loop(0, CH, body, 0)

    def start_in(t, j):
        b = t // N_CH
        k = t % N_CH
        return pltpu.async_copy(
            x_hbm.at[b, pl.ds(s0 + k * CH, CH)], buf_v.at[j], sem_in.at[j]
        )

    def start_out(t, j):
        b = t // N_CH
        k = t % N_CH
        return pltpu.async_copy(
            buf_v.at[j], out_hbm.at[b, pl.ds(s0 + k * CH, CH)], sem_out.at[j]
        )

    in_h = [None, None]
    out_h = [None, None]
    for t in range(T):
        j = t % 2
        if out_h[j] is not None:
            out_h[j].wait()
        in_h[j] = start_in(t, j)
        if t >= 1:
            jp = (t - 1) % 2
            in_h[jp].wait()
            add_chunk(buf_v.at[jp], (t - 1) % N_CH)
            out_h[jp] = start_out(t - 1, jp)
    jl = (T - 1) % 2
    in_h[jl].wait()
    add_chunk(buf_v.at[jl], (T - 1) % N_CH)
    out_h[jl] = start_out(T - 1, jl)
    out_h[(T - 2) % 2].wait()
    out_h[jl].wait()


def kernel(x, pos_weight):
    mesh = plsc.VectorSubcoreMesh(core_axis_name="c", subcore_axis_name="s")
    k = functools.partial(
        pl.kernel,
        mesh=mesh,
        out_type=jax.ShapeDtypeStruct((B, S, D), jnp.float32),
        scratch_types=[
            pltpu.VMEM((S_PER_W, D), jnp.float32),
            pltpu.VMEM((2, CH, D), jnp.float32),
            pltpu.SemaphoreType.DMA((2,)),
            pltpu.SemaphoreType.DMA((2,)),
        ],
    )(_sc_body)
    return k(x, pos_weight)


# speedup vs baseline: 4.0653x; 2.2755x over previous
import functools

import jax
import jax.numpy as jnp
from jax import lax
from jax.experimental import pallas as pl
from jax.experimental.pallas import tpu as pltpu
from jax.experimental.pallas import tpu_sc as plsc

B, S, D = 4, 2048, 1024
NW = 32
S_PER_W = S // NW
CH = 16
N_CH = S_PER_W // CH
T = B * N_CH


def _sc_body(x_hbm, pos_hbm, out_hbm, pos_v, buf_v, sem_in, sem_out):
    wid = lax.axis_index("s") * 2 + lax.axis_index("c")
    s0 = wid * S_PER_W
    pltpu.sync_copy(pos_hbm.at[pl.ds(s0, S_PER_W)], pos_v)

    def add_chunk(buf, k):
        def body(r, _):
            pr = k * CH + r

            @plsc.parallel_loop(0, D, step=16, unroll=8)
            def _col(c):
                buf[r, pl.ds(c, 16)] = buf[r, pl.ds(c, 16)] + pos_v[
                    pr, pl.ds(c, 16)
                ]

            return 0

        lax.fori_loop(0, CH, body, 0)

    def start_in(t, j):
        b = t // N_CH
        k = t % N_CH
        return pltpu.async_copy(
            x_hbm.at[b, pl.ds(s0 + k * CH, CH)], buf_v.at[j], sem_in.at[j]
        )

    def start_out(t, j):
        b = t // N_CH
        k = t % N_CH
        return pltpu.async_copy(
            buf_v.at[j], out_hbm.at[b, pl.ds(s0 + k * CH, CH)], sem_out.at[j]
        )

    in_h = [None, None]
    out_h = [None, None]
    for t in range(T):
        j = t % 2
        if out_h[j] is not None:
            out_h[j].wait()
        in_h[j] = start_in(t, j)
        if t >= 1:
            jp = (t - 1) % 2
            in_h[jp].wait()
            add_chunk(buf_v.at[jp], (t - 1) % N_CH)
            out_h[jp] = start_out(t - 1, jp)
    jl = (T - 1) % 2
    in_h[jl].wait()
    add_chunk(buf_v.at[jl], (T - 1) % N_CH)
    out_h[jl] = start_out(T - 1, jl)
    out_h[(T - 2) % 2].wait()
    out_h[jl].wait()


def kernel(x, pos_weight):
    mesh = plsc.VectorSubcoreMesh(core_axis_name="c", subcore_axis_name="s")
    k = functools.partial(
        pl.kernel,
        mesh=mesh,
        out_type=jax.ShapeDtypeStruct((B, S, D), jnp.float32),
        scratch_types=[
            pltpu.VMEM((S_PER_W, D), jnp.float32),
            pltpu.VMEM((2, CH, D), jnp.float32),
            pltpu.SemaphoreType.DMA((2,)),
            pltpu.SemaphoreType.DMA((2,)),
        ],
    )(_sc_body)
    return k(x, pos_weight)


# speedup vs baseline: 9.9193x; 2.4400x over previous
import jax
import jax.numpy as jnp
from jax.experimental import pallas as pl


def _add_kernel(x_ref, pos_ref, o_ref):
    o_ref[...] = x_ref[...] + pos_ref[...]


def kernel(x, pos_weight):
    B, S, D = x.shape
    BS = 2048
    grid = (S // BS, B)
    return pl.pallas_call(
        _add_kernel,
        grid=grid,
        in_specs=[
            pl.BlockSpec((1, BS, D), lambda s, b: (b, s, 0)),
            pl.BlockSpec((BS, D), lambda s, b: (s, 0)),
        ],
        out_specs=pl.BlockSpec((1, BS, D), lambda s, b: (b, s, 0)),
        out_shape=jax.ShapeDtypeStruct(x.shape, x.dtype),
    )(x, pos_weight)


# speedup vs baseline: 9.9619x; 1.0043x over previous
import jax
import jax.numpy as jnp
from jax.experimental import pallas as pl


def _add_kernel(x_ref, pos_ref, o_ref):
    o_ref[...] = x_ref[...] + pos_ref[...]


def kernel(x, pos_weight):
    B, S, D = x.shape
    BS = S
    grid = (S // BS, B)
    return pl.pallas_call(
        _add_kernel,
        grid=grid,
        in_specs=[
            pl.BlockSpec((1, BS, D), lambda s, b: (b, s, 0)),
            pl.BlockSpec((BS, D), lambda s, b: (s, 0)),
        ],
        out_specs=pl.BlockSpec((1, BS, D), lambda s, b: (b, s, 0)),
        out_shape=jax.ShapeDtypeStruct(x.shape, x.dtype),
    )(x, pos_weight)
